# Initial kernel scaffold; baseline (speedup 1.0000x reference)
import jax
import jax.numpy as jnp
from jax.experimental import pallas as pl


def kernel(input_ids, weight, position_embedding):
    raise NotImplementedError("write your pallas kernel here")


# speedup vs baseline: 3.3543x; 3.3543x over previous
import functools

import jax
import jax.numpy as jnp
from jax import lax
from jax.experimental import pallas as pl
from jax.experimental.pallas import tpu as pltpu
from jax.experimental.pallas import tpu_sc as plsc

B, S, D = 4096, 77, 128
LANES = 16

_info = plsc.get_sparse_core_info()
NC, NS = _info.num_cores, _info.num_subcores
NW = NC * NS
SEQ_PER_W = B // NW


def _emb_body(ids_hbm, w_hbm, pos_hbm, out_hbm, idx_v, rows_v, pos_v, sem):
    wid = lax.axis_index("s") * NC + lax.axis_index("c")
    seq0 = wid * SEQ_PER_W
    pltpu.sync_copy(pos_hbm, pos_v)
    pltpu.sync_copy(ids_hbm.at[pl.ds(seq0, SEQ_PER_W), :], idx_v)

    def chunk(ci, carry):
        pltpu.async_copy(w_hbm.at[idx_v.at[ci]], rows_v, sem).wait()

        def row(r, c2):
            for c in range(D // LANES):
                sl = pl.ds(c * LANES, LANES)
                rows_v[r, sl] = rows_v[r, sl] + pos_v[r, sl]
            return c2

        lax.fori_loop(0, S, row, 0)
        pltpu.sync_copy(rows_v, out_hbm.at[seq0 + ci])
        return carry

    lax.fori_loop(0, SEQ_PER_W, chunk, 0)


@jax.jit
def kernel(input_ids, weight, position_embedding):
    mesh = plsc.VectorSubcoreMesh(core_axis_name="c", subcore_axis_name="s")
    out = pl.kernel(
        _emb_body,
        mesh=mesh,
        out_type=jax.ShapeDtypeStruct((B, S, D), jnp.float32),
        scratch_types=[
            pltpu.VMEM((SEQ_PER_W, S), jnp.int32),
            pltpu.VMEM((S, D), jnp.float32),
            pltpu.VMEM((S, D), jnp.float32),
            pltpu.SemaphoreType.DMA,
        ],
    )(input_ids.astype(jnp.int32), weight, position_embedding)
    return out


# speedup vs baseline: 5.3530x; 1.5959x over previous
import jax
import jax.numpy as jnp
from jax import lax
from jax.experimental import pallas as pl
from jax.experimental.pallas import tpu as pltpu
from jax.experimental.pallas import tpu_sc as plsc

B, S, D = 4096, 77, 128
LANES = 16
NB = 4

_info = plsc.get_sparse_core_info()
NC, NS = _info.num_cores, _info.num_subcores
NW = NC * NS
SEQ_PER_W = B // NW


def _emb_body(ids_hbm, w_hbm, pos_hbm, out_hbm, idx_v, rows_v, pos_v, gsem, ssem):
    wid = lax.axis_index("s") * NC + lax.axis_index("c")
    seq0 = wid * SEQ_PER_W
    pltpu.sync_copy(pos_hbm, pos_v)
    pltpu.sync_copy(ids_hbm.at[pl.ds(seq0, SEQ_PER_W), :], idx_v)

    def g_copy(ci, b):
        return pltpu.make_async_copy(
            w_hbm.at[idx_v.at[ci]], rows_v.at[b], gsem.at[b])

    def s_copy(ci, b):
        return pltpu.make_async_copy(
            rows_v.at[b], out_hbm.at[seq0 + ci], ssem.at[b])

    g_copy(0, 0).start()

    def outer(g, carry):
        for b in range(NB):
            ci = g * NB + b
            nb = (b + 1) % NB

            @pl.when(ci + 1 < SEQ_PER_W)
            def _():
                @pl.when(ci >= NB - 1)
                def _():
                    s_copy(ci - (NB - 1), nb).wait()
                g_copy(ci + 1, nb).start()

            g_copy(ci, b).wait()

            def row(r, c2):
                for c in range(D // LANES):
                    sl = pl.ds(c * LANES, LANES)
                    rows_v[b, r, sl] = rows_v[b, r, sl] + pos_v[r, sl]
                return c2

            lax.fori_loop(0, S, row, 0)
            s_copy(ci, b).start()
        return carry

    lax.fori_loop(0, SEQ_PER_W // NB, outer, 0)
    for b in range(NB):
        s_copy(SEQ_PER_W - NB + b, b).wait()


@jax.jit
def kernel(input_ids, weight, position_embedding):
    mesh = plsc.VectorSubcoreMesh(core_axis_name="c", subcore_axis_name="s")
    out = pl.kernel(
        _emb_body,
        mesh=mesh,
        out_type=jax.ShapeDtypeStruct((B, S, D), jnp.float32),
        scratch_types=[
            pltpu.VMEM((SEQ_PER_W, S), jnp.int32),
            pltpu.VMEM((NB, S, D), jnp.float32),
            pltpu.VMEM((S, D), jnp.float32),
            pltpu.SemaphoreType.DMA((NB,)),
            pltpu.SemaphoreType.DMA((NB,)),
        ],
    )(input_ids.astype(jnp.int32), weight, position_embedding)
    return out


# speedup vs baseline: 5.7463x; 1.0735x over previous
import jax
import jax.numpy as jnp
from jax import lax
from jax.experimental import pallas as pl
from jax.experimental.pallas import tpu as pltpu
from jax.experimental.pallas import tpu_sc as plsc

B, S, D = 4096, 77, 128
LANES = 16
NB = 8
GA = 2
RU = 7

_info = plsc.get_sparse_core_info()
NC, NS = _info.num_cores, _info.num_subcores
NW = NC * NS
SEQ_PER_W = B // NW


def _emb_body(ids_hbm, w_hbm, pos_hbm, out_hbm, idx_v, rows_v, pos_v, gsem, ssem):
    wid = lax.axis_index("s") * NC + lax.axis_index("c")
    seq0 = wid * SEQ_PER_W
    pltpu.sync_copy(pos_hbm, pos_v)
    pltpu.sync_copy(ids_hbm.at[pl.ds(seq0, SEQ_PER_W), :], idx_v)

    def g_copy(ci, b):
        return pltpu.make_async_copy(
            w_hbm.at[idx_v.at[ci]], rows_v.at[b], gsem.at[b])

    def s_copy(ci, b):
        return pltpu.make_async_copy(
            rows_v.at[b], out_hbm.at[seq0 + ci], ssem.at[b])

    for a in range(GA):
        g_copy(a, a).start()

    def outer(g, carry):
        for b in range(NB):
            ci = g * NB + b
            nb = (b + GA) % NB

            @pl.when(ci + GA < SEQ_PER_W)
            def _():
                @pl.when(ci >= NB - GA)
                def _():
                    s_copy(ci + GA - NB, nb).wait()
                g_copy(ci + GA, nb).start()

            g_copy(ci, b).wait()

            def row(rr, c2):
                for k in range(RU):
                    r = rr * RU + k
                    for c in range(D // LANES):
                        sl = pl.ds(c * LANES, LANES)
                        rows_v[b, r, sl] = rows_v[b, r, sl] + pos_v[r, sl]
                return c2

            lax.fori_loop(0, S // RU, row, 0)
            s_copy(ci, b).start()
        return carry

    lax.fori_loop(0, SEQ_PER_W // NB, outer, 0)
    for b in range(NB):
        s_copy(SEQ_PER_W - NB + b, b).wait()


@jax.jit
def kernel(input_ids, weight, position_embedding):
    mesh = plsc.VectorSubcoreMesh(core_axis_name="c", subcore_axis_name="s")
    out = pl.kernel(
        _emb_body,
        mesh=mesh,
        out_type=jax.ShapeDtypeStruct((B, S, D), jnp.float32),
        scratch_types=[
            pltpu.VMEM((SEQ_PER_W, S), jnp.int32),
            pltpu.VMEM((NB, S, D), jnp.float32),
            pltpu.VMEM((S, D), jnp.float32),
            pltpu.SemaphoreType.DMA((NB,)),
            pltpu.SemaphoreType.DMA((NB,)),
        ],
    )(input_ids.astype(jnp.int32), weight, position_embedding)
    return out


# speedup vs baseline: 5.7511x; 1.0008x over previous
import jax
import jax.numpy as jnp
from jax import lax
from jax.experimental import pallas as pl
from jax.experimental.pallas import tpu as pltpu
from jax.experimental.pallas import tpu_sc as plsc

B, S, D = 4096, 77, 128
LANES = 16
NB = 8
GA = 2
RU = 7

_info = plsc.get_sparse_core_info()
NC, NS = _info.num_cores, _info.num_subcores
NW = NC * NS
SEQ_PER_W = B // NW


def _emb_body(ids_hbm, w_hbm, pos_hbm, out_hbm, idx_v, rows_v, pos_v, gsem, ssem):
    wid = lax.axis_index("s") * NC + lax.axis_index("c")
    seq0 = wid * SEQ_PER_W
    pltpu.sync_copy(pos_hbm, pos_v)
    pltpu.sync_copy(ids_hbm.at[pl.ds(seq0, SEQ_PER_W), :], idx_v)

    def g_copy(ci, b):
        return pltpu.make_async_copy(
            w_hbm.at[idx_v.at[ci]], rows_v.at[b], gsem.at[b])

    def s_copy(ci, b):
        return pltpu.make_async_copy(
            rows_v.at[b], out_hbm.at[seq0 + ci], ssem.at[b])

    for a in range(GA):
        g_copy(a, a).start()

    def outer(g, carry):
        for b in range(NB):
            ci = g * NB + b
            nb = (b + GA) % NB

            @pl.when(ci + GA < SEQ_PER_W)
            def _():
                @pl.when(ci >= NB - GA)
                def _():
                    s_copy(ci + GA - NB, nb).wait()
                g_copy(ci + GA, nb).start()

            g_copy(ci, b).wait()

            def row(rr, c2):
                for k in range(RU):
                    r = rr * RU + k
                    for c in range(D // LANES):
                        sl = pl.ds(c * LANES, LANES)
                        rows_v[b, r, sl] = rows_v[b, r, sl] + pos_v[r, sl]
                return c2

            lax.fori_loop(0, S // RU, row, 0)
            s_copy(ci, b).start()
        return carry

    lax.fori_loop(0, SEQ_PER_W // NB, outer, 0)
    for b in range(NB):
        s_copy(SEQ_PER_W - NB + b, b).wait()


@jax.jit
def kernel(input_ids, weight, position_embedding):
    mesh = plsc.VectorSubcoreMesh(core_axis_name="c", subcore_axis_name="s")
    out = pl.kernel(
        _emb_body,
        mesh=mesh,
        compiler_params=pltpu.CompilerParams(use_tc_tiling_on_sc=True),
        out_type=jax.ShapeDtypeStruct((B, S, D), jnp.float32),
        scratch_types=[
            pltpu.VMEM((SEQ_PER_W, S), jnp.int32),
            pltpu.VMEM((NB, S, D), jnp.float32),
            pltpu.VMEM((S, D), jnp.float32),
            pltpu.SemaphoreType.DMA((NB,)),
            pltpu.SemaphoreType.DMA((NB,)),
        ],
    )(input_ids.astype(jnp.int32), weight, position_embedding)
    return out
